# Initial kernel scaffold; baseline (speedup 1.0000x reference)
import jax
import jax.numpy as jnp
from jax.experimental import pallas as pl


def kernel(user_feat_batch, item_feat_batch, age_table, user_tables, item_tables, uW1, ub1, uW2, ub2, iW1, ib1, iW2, ib2):
    raise NotImplementedError("write your pallas kernel here")


# speedup vs baseline: 1.8661x; 1.8661x over previous
import functools

import jax
import jax.numpy as jnp
from jax import lax
from jax.experimental import pallas as pl
from jax.experimental.pallas import tpu as pltpu
from jax.experimental.pallas import tpu_sc as plsc

B = 16384
D = 32
V = 100000
NU = 13
NI = 13

_info = plsc.get_sparse_core_info()
_NC, _NS = _info.num_cores, _info.num_subcores
_NW = _NC * _NS
_BPW = B // _NW


def _sc_gather(age_table, user_flat, item_flat, idx_age, idx_user, idx_item):
    mesh = plsc.VectorSubcoreMesh(core_axis_name="c", subcore_axis_name="s")

    @functools.partial(
        pl.kernel,
        mesh=mesh,
        compiler_params=pltpu.CompilerParams(use_tc_tiling_on_sc=False),
        out_type=[
            jax.ShapeDtypeStruct((1 + NU, B, D), jnp.float32),
            jax.ShapeDtypeStruct((NI, B, D), jnp.float32),
        ],
        scratch_types=[
            pltpu.VMEM((_BPW,), jnp.int32),
            pltpu.VMEM((_BPW, D), jnp.float32),
            pltpu.SemaphoreType.DMA,
        ],
    )
    def gather_kernel(age_hbm, u_hbm, i_hbm, ia_hbm, iu_hbm, ii_hbm,
                      xu_hbm, xi_hbm, idx_v, rows_v, sem):
        wid = lax.axis_index("s") * _NC + lax.axis_index("c")
        base = wid * _BPW

        def one_feature(idx_hbm, idx_off, table_hbm, out_hbm, feat):
            pltpu.sync_copy(idx_hbm.at[pl.ds(idx_off, _BPW)], idx_v)
            pltpu.async_copy(table_hbm.at[idx_v], rows_v, sem).wait()
            pltpu.sync_copy(rows_v, out_hbm.at[feat, pl.ds(base, _BPW)])

        one_feature(ia_hbm, base, age_hbm, xu_hbm, 0)
        for j in range(NU):
            one_feature(iu_hbm, j * B + base, u_hbm, xu_hbm, 1 + j)
        for j in range(NI):
            one_feature(ii_hbm, j * B + base, i_hbm, xi_hbm, j)

    return gather_kernel(age_table, user_flat, item_flat,
                         idx_age, idx_user, idx_item)


def _mlp_body(xu_ref, xi_ref, uW1_ref, ub1_ref, uW2_ref, ub2_ref,
              iW1_ref, ib1_ref, iW2_ref, ib2_ref, zu_ref, zi_ref):
    def tower(x_ref, nf, W1_ref, b1, W2_ref, b2):
        h = jnp.dot(x_ref[0], W1_ref[0],
                    preferred_element_type=jnp.float32)
        for j in range(1, nf):
            h = h + jnp.dot(x_ref[j], W1_ref[j],
                            preferred_element_type=jnp.float32)
        h = jnp.maximum(h + b1, 0.0)
        z = jnp.dot(h, W2_ref[...], preferred_element_type=jnp.float32) + b2
        n = jnp.sqrt(jnp.sum(z * z, axis=1, keepdims=True))
        return z / jnp.maximum(n, 1e-12)

    zu_ref[...] = tower(xu_ref, 1 + NU, uW1_ref, ub1_ref[...],
                        uW2_ref, ub2_ref[...])
    zi_ref[...] = tower(xi_ref, NI, iW1_ref, ib1_ref[...],
                        iW2_ref, ib2_ref[...])


def _tc_mlp(xu, xi, uW1, ub1, uW2, ub2, iW1, ib1, iW2, ib2, bs=1024):
    grid = (B // bs,)
    full2 = lambda shape: pl.BlockSpec(shape, lambda i: (0, 0))
    full3 = lambda shape: pl.BlockSpec(shape, lambda i: (0, 0, 0))
    return pl.pallas_call(
        _mlp_body,
        grid=grid,
        compiler_params=pltpu.CompilerParams(
            dimension_semantics=("arbitrary",),
            vmem_limit_bytes=100 * 1024 * 1024,
        ),
        in_specs=[
            pl.BlockSpec((1 + NU, bs, D), lambda i: (0, i, 0)),
            pl.BlockSpec((NI, bs, D), lambda i: (0, i, 0)),
            full3(((1 + NU), D, D)),
            full2((1, D)),
            full2((D, D)),
            full2((1, D)),
            full3((NI, D, D)),
            full2((1, D)),
            full2((D, D)),
            full2((1, D)),
        ],
        out_specs=[
            pl.BlockSpec((bs, D), lambda i: (i, 0)),
            pl.BlockSpec((bs, D), lambda i: (i, 0)),
        ],
        out_shape=[
            jax.ShapeDtypeStruct((B, D), jnp.float32),
            jax.ShapeDtypeStruct((B, D), jnp.float32),
        ],
    )(xu, xi, uW1.reshape(1 + NU, D, D), ub1.reshape(1, D), uW2,
      ub2.reshape(1, D), iW1.reshape(NI, D, D), ib1.reshape(1, D), iW2,
      ib2.reshape(1, D))


def kernel(user_feat_batch, item_feat_batch, age_table, user_tables,
           item_tables, uW1, ub1, uW2, ub2, iW1, ib1, iW2, ib2):
    offs = (jnp.arange(NU, dtype=jnp.int32) * V)[:, None]
    idx_age = user_feat_batch[:, 0]
    idx_user = (user_feat_batch[:, 1:].T + offs).reshape(-1)
    idx_item = (item_feat_batch.T + offs).reshape(-1)
    user_flat = user_tables.reshape(NU * V, D)
    item_flat = item_tables.reshape(NI * V, D)

    xu, xi = _sc_gather(age_table, user_flat, item_flat,
                        idx_age, idx_user, idx_item)
    return _tc_mlp(xu, xi, uW1, ub1, uW2, ub2, iW1, ib1, iW2, ib2)


# speedup vs baseline: 1.9121x; 1.0247x over previous
import functools

import jax
import jax.numpy as jnp
from jax import lax
from jax.experimental import pallas as pl
from jax.experimental.pallas import tpu as pltpu
from jax.experimental.pallas import tpu_sc as plsc

B = 16384
D = 32
V = 100000
NU = 13
NI = 13

_info = plsc.get_sparse_core_info()
_NC, _NS = _info.num_cores, _info.num_subcores
_NW = _NC * _NS
_BPW = B // _NW


_NF = 1 + NU + NI
_NBUF = 4


def _sc_gather(age_table, user_flat, item_flat, idx_all):
    mesh = plsc.VectorSubcoreMesh(core_axis_name="c", subcore_axis_name="s")

    @functools.partial(
        pl.kernel,
        mesh=mesh,
        compiler_params=pltpu.CompilerParams(use_tc_tiling_on_sc=False),
        out_type=[
            jax.ShapeDtypeStruct((1 + NU, B, D), jnp.float32),
            jax.ShapeDtypeStruct((NI, B, D), jnp.float32),
        ],
        scratch_types=[
            pltpu.VMEM((_NF, _BPW), jnp.int32),
            [pltpu.VMEM((_BPW, D), jnp.float32) for _ in range(_NBUF)],
            [pltpu.SemaphoreType.DMA for _ in range(_NBUF)],
            [pltpu.SemaphoreType.DMA for _ in range(_NBUF)],
        ],
    )
    def gather_kernel(age_hbm, u_hbm, i_hbm, idx_hbm,
                      xu_hbm, xi_hbm, idx_v, rows, gsem, wsem):
        wid = lax.axis_index("s") * _NC + lax.axis_index("c")
        base = wid * _BPW

        pltpu.sync_copy(idx_hbm.at[wid], idx_v)

        def table_of(j):
            return age_hbm if j == 0 else (u_hbm if j <= NU else i_hbm)

        def out_of(j):
            if j <= NU:
                return xu_hbm.at[j, pl.ds(base, _BPW)]
            return xi_hbm.at[j - 1 - NU, pl.ds(base, _BPW)]

        gh = [None] * _NF
        wh = [None] * _NF

        def start_write(jj):
            gh[jj].wait()
            wh[jj] = pltpu.async_copy(rows[jj % _NBUF], out_of(jj),
                                      wsem[jj % _NBUF])

        for j in range(_NF):
            b = j % _NBUF
            if j >= _NBUF:
                wh[j - _NBUF].wait()
            gh[j] = pltpu.async_copy(table_of(j).at[idx_v.at[j]],
                                     rows[b], gsem[b])
            if j >= _NBUF - 1:
                start_write(j - (_NBUF - 1))
        for jj in range(_NF - (_NBUF - 1), _NF):
            start_write(jj)
        for jj in range(_NF - _NBUF, _NF):
            wh[jj].wait()

    return gather_kernel(age_table, user_flat, item_flat, idx_all)


def _mlp_body(xu_ref, xi_ref, uW1_ref, ub1_ref, uW2_ref, ub2_ref,
              iW1_ref, ib1_ref, iW2_ref, ib2_ref, zu_ref, zi_ref):
    def tower(x_ref, nf, W1_ref, b1, W2_ref, b2):
        h = jnp.dot(x_ref[0], W1_ref[0],
                    preferred_element_type=jnp.float32)
        for j in range(1, nf):
            h = h + jnp.dot(x_ref[j], W1_ref[j],
                            preferred_element_type=jnp.float32)
        h = jnp.maximum(h + b1, 0.0)
        z = jnp.dot(h, W2_ref[...], preferred_element_type=jnp.float32) + b2
        n = jnp.sqrt(jnp.sum(z * z, axis=1, keepdims=True))
        return z / jnp.maximum(n, 1e-12)

    zu_ref[...] = tower(xu_ref, 1 + NU, uW1_ref, ub1_ref[...],
                        uW2_ref, ub2_ref[...])
    zi_ref[...] = tower(xi_ref, NI, iW1_ref, ib1_ref[...],
                        iW2_ref, ib2_ref[...])


def _tc_mlp(xu, xi, uW1, ub1, uW2, ub2, iW1, ib1, iW2, ib2, bs=1024):
    grid = (B // bs,)
    full2 = lambda shape: pl.BlockSpec(shape, lambda i: (0, 0))
    full3 = lambda shape: pl.BlockSpec(shape, lambda i: (0, 0, 0))
    return pl.pallas_call(
        _mlp_body,
        grid=grid,
        compiler_params=pltpu.CompilerParams(
            dimension_semantics=("arbitrary",),
            vmem_limit_bytes=100 * 1024 * 1024,
        ),
        in_specs=[
            pl.BlockSpec((1 + NU, bs, D), lambda i: (0, i, 0)),
            pl.BlockSpec((NI, bs, D), lambda i: (0, i, 0)),
            full3(((1 + NU), D, D)),
            full2((1, D)),
            full2((D, D)),
            full2((1, D)),
            full3((NI, D, D)),
            full2((1, D)),
            full2((D, D)),
            full2((1, D)),
        ],
        out_specs=[
            pl.BlockSpec((bs, D), lambda i: (i, 0)),
            pl.BlockSpec((bs, D), lambda i: (i, 0)),
        ],
        out_shape=[
            jax.ShapeDtypeStruct((B, D), jnp.float32),
            jax.ShapeDtypeStruct((B, D), jnp.float32),
        ],
    )(xu, xi, uW1.reshape(1 + NU, D, D), ub1.reshape(1, D), uW2,
      ub2.reshape(1, D), iW1.reshape(NI, D, D), ib1.reshape(1, D), iW2,
      ib2.reshape(1, D))


def kernel(user_feat_batch, item_feat_batch, age_table, user_tables,
           item_tables, uW1, ub1, uW2, ub2, iW1, ib1, iW2, ib2):
    offs = (jnp.arange(NU, dtype=jnp.int32) * V)[:, None]
    idx_user = user_feat_batch[:, 1:].T + offs
    idx_item = item_feat_batch.T + offs
    idx_all = jnp.concatenate(
        [user_feat_batch[:, :1].T, idx_user, idx_item], axis=0)
    idx_all = idx_all.reshape(_NF, _NW, _BPW).transpose(1, 0, 2)
    user_flat = user_tables.reshape(NU * V, D)
    item_flat = item_tables.reshape(NI * V, D)

    xu, xi = _sc_gather(age_table, user_flat, item_flat, idx_all)
    return _tc_mlp(xu, xi, uW1, ub1, uW2, ub2, iW1, ib1, iW2, ib2)


# speedup vs baseline: 2.5135x; 1.3145x over previous
import functools

import jax
import jax.numpy as jnp
from jax import lax
from jax.experimental import pallas as pl
from jax.experimental.pallas import tpu as pltpu
from jax.experimental.pallas import tpu_sc as plsc

B = 16384
D = 32
V = 100000
NU = 13
NI = 13
_NF = 1 + NU + NI

_info = plsc.get_sparse_core_info()
_NC, _NS = _info.num_cores, _info.num_subcores
_NW = _NC * _NS
_CHUNK = 2048
_NCHUNK = B // _CHUNK


def _sc_gather_t(age_t, user_t, item_t, idx_flat):
    mesh = plsc.VectorSubcoreMesh(core_axis_name="c", subcore_axis_name="s")

    @functools.partial(
        pl.kernel,
        mesh=mesh,
        compiler_params=pltpu.CompilerParams(use_tc_tiling_on_sc=False,
                                             needs_layout_passes=False),
        out_type=[
            jax.ShapeDtypeStruct(((1 + NU) * D * B,), jnp.float32),
            jax.ShapeDtypeStruct((NI * D * B,), jnp.float32),
        ],
        scratch_types=[
            pltpu.VMEM((V,), jnp.float32),
            pltpu.VMEM((_CHUNK,), jnp.int32),
            pltpu.VMEM((_CHUNK,), jnp.float32),
            pltpu.SemaphoreType.DMA,
        ],
    )
    def gather_kernel(age_hbm, u_hbm, i_hbm, idx_hbm,
                      xu_hbm, xi_hbm, trow_v, idx_v, outc_v, sem):
        d = lax.axis_index("s") * _NC + lax.axis_index("c")

        def gather_chunk(i, _):
            iv = idx_v[pl.ds(i * 16, 16)]
            outc_v[pl.ds(i * 16, 16)] = plsc.load_gather(trow_v, [iv])
            return 0

        for f in range(_NF):
            if f == 0:
                row_ref = age_hbm.at[d]
            elif f <= NU:
                row_ref = u_hbm.at[(f - 1) * D + d]
            else:
                row_ref = i_hbm.at[(f - 1 - NU) * D + d]
            pltpu.sync_copy(row_ref, trow_v)
            if f <= NU:
                out_hbm = xu_hbm
                obase = (f * D + d) * B
            else:
                out_hbm = xi_hbm
                obase = ((f - 1 - NU) * D + d) * B
            for c in range(_NCHUNK):
                pltpu.sync_copy(
                    idx_hbm.at[pl.ds(f * B + c * _CHUNK, _CHUNK)], idx_v)
                lax.fori_loop(0, _CHUNK // 16, gather_chunk, 0)
                pltpu.sync_copy(outc_v,
                                out_hbm.at[pl.ds(obase + c * _CHUNK, _CHUNK)])

    return gather_kernel(age_t, user_t, item_t, idx_flat)


def _mlp_body(xu_ref, xi_ref, uW1t_ref, ub1_ref, uW2t_ref, ub2_ref,
              iW1t_ref, ib1_ref, iW2t_ref, ib2_ref, zu_ref, zi_ref):
    def tower(xt, W1t, b1, W2t, b2):
        h = jnp.maximum(
            jnp.dot(W1t, xt, preferred_element_type=jnp.float32) + b1, 0.0)
        z = jnp.dot(W2t, h, preferred_element_type=jnp.float32) + b2
        n = jnp.sqrt(jnp.sum(z * z, axis=0, keepdims=True))
        return z / jnp.maximum(n, 1e-12)

    zu_ref[...] = tower(xu_ref[...], uW1t_ref[...], ub1_ref[...],
                        uW2t_ref[...], ub2_ref[...])
    zi_ref[...] = tower(xi_ref[...], iW1t_ref[...], ib1_ref[...],
                        iW2t_ref[...], ib2_ref[...])


def _tc_mlp_t(xu_t, xi_t, uW1t, ub1, uW2t, ub2, iW1t, ib1, iW2t, ib2,
              bs=2048):
    grid = (B // bs,)
    full = lambda shape: pl.BlockSpec(shape, lambda i: (0, 0))
    zu_t, zi_t = pl.pallas_call(
        _mlp_body,
        grid=grid,
        compiler_params=pltpu.CompilerParams(
            dimension_semantics=("arbitrary",),
            vmem_limit_bytes=100 * 1024 * 1024,
        ),
        in_specs=[
            pl.BlockSpec(((1 + NU) * D, bs), lambda i: (0, i)),
            pl.BlockSpec((NI * D, bs), lambda i: (0, i)),
            full((D, (1 + NU) * D)),
            full((D, 1)),
            full((D, D)),
            full((D, 1)),
            full((D, NI * D)),
            full((D, 1)),
            full((D, D)),
            full((D, 1)),
        ],
        out_specs=[
            pl.BlockSpec((D, bs), lambda i: (0, i)),
            pl.BlockSpec((D, bs), lambda i: (0, i)),
        ],
        out_shape=[
            jax.ShapeDtypeStruct((D, B), jnp.float32),
            jax.ShapeDtypeStruct((D, B), jnp.float32),
        ],
    )(xu_t, xi_t, uW1t, ub1.reshape(D, 1), uW2t, ub2.reshape(D, 1),
      iW1t, ib1.reshape(D, 1), iW2t, ib2.reshape(D, 1))
    return zu_t.T, zi_t.T


def kernel(user_feat_batch, item_feat_batch, age_table, user_tables,
           item_tables, uW1, ub1, uW2, ub2, iW1, ib1, iW2, ib2):
    idx_flat = jnp.concatenate(
        [user_feat_batch.T, item_feat_batch.T], axis=0).reshape(-1)
    age_t = age_table.T
    user_t = user_tables.transpose(0, 2, 1).reshape(NU * D, V)
    item_t = item_tables.transpose(0, 2, 1).reshape(NI * D, V)

    xu_flat, xi_flat = _sc_gather_t(age_t, user_t, item_t, idx_flat)
    xu_t = xu_flat.reshape((1 + NU) * D, B)
    xi_t = xi_flat.reshape(NI * D, B)
    return _tc_mlp_t(xu_t, xi_t, uW1.T, ub1, uW2.T, ub2,
                     iW1.T, ib1, iW2.T, ib2)
